# baseline (speedup 1.0000x reference)
import functools

import jax
import jax.numpy as jnp
from jax import lax
from jax.experimental import pallas as pl
from jax.experimental.pallas import tpu as pltpu
from jax.experimental.pallas import tpu_sc as plsc

_VOCAB = 50257
_EMBED = 1024
_MAX_SEQ = 2048
_BATCH = 4
_ROWS = _BATCH * _MAX_SEQ
_NC = 2
_NS = 16
_NW = _NC * _NS
_ROWS_PER_W = _ROWS // _NW
_CHUNK = 32
_NCHUNK = _ROWS_PER_W // _CHUNK
_LANES = 16


def _emb_body(ids_hbm, tok_hbm, pos_hbm, out_hbm, idx_v, tokbuf, posbuf, sem):
    wid = lax.axis_index("s") * _NC + lax.axis_index("c")
    base = wid * _ROWS_PER_W
    pos0 = lax.rem(base, _MAX_SEQ)

    pltpu.sync_copy(ids_hbm.at[wid], idx_v)

    for c in range(_NCHUNK):
        row0 = base + c * _CHUNK
        pltpu.async_copy(tok_hbm.at[idx_v.at[c]], tokbuf, sem).wait()
        pltpu.sync_copy(pos_hbm.at[pl.ds(pos0 + c * _CHUNK, _CHUNK)], posbuf)

        def col_body(j, _):
            s = pl.ds(j * _LANES, _LANES)

            def row_body(r, _):
                tokbuf[r, s] = tokbuf[r, s] + posbuf[r, s]
                return 0

            return lax.fori_loop(0, _CHUNK, row_body, 0)

        lax.fori_loop(0, _EMBED // _LANES, col_body, 0)

        pltpu.sync_copy(tokbuf, out_hbm.at[pl.ds(row0, _CHUNK)])


@jax.jit
def _embed(ids, tok_table, pos_table):
    mesh = plsc.VectorSubcoreMesh(core_axis_name="c", subcore_axis_name="s")
    run = functools.partial(
        pl.kernel,
        out_type=jax.ShapeDtypeStruct((_ROWS, _EMBED), jnp.float32),
        mesh=mesh,
        scratch_types=[
            pltpu.VMEM((_NCHUNK, _CHUNK), jnp.int32),
            pltpu.VMEM((_CHUNK, _EMBED), jnp.float32),
            pltpu.VMEM((_CHUNK, _EMBED), jnp.float32),
            pltpu.SemaphoreType.DMA,
        ],
    )(_emb_body)
    return run(ids, tok_table, pos_table)


def kernel(input_ids, token_table, pos_table):
    ids = input_ids.astype(jnp.int32).reshape(_NW, _NCHUNK, _CHUNK)
    out = _embed(ids, token_table, pos_table)
    return out.reshape(_BATCH, _MAX_SEQ, _EMBED)


# speedup vs baseline: 2.3064x; 2.3064x over previous
import functools

import jax
import jax.numpy as jnp
from jax import lax
from jax.experimental import pallas as pl
from jax.experimental.pallas import tpu as pltpu
from jax.experimental.pallas import tpu_sc as plsc

_VOCAB = 50257
_EMBED = 1024
_MAX_SEQ = 2048
_BATCH = 4
_NC = 2
_NS = 16
_NW = _NC * _NS
_SEQ_PER_W = _MAX_SEQ // _NW
_CHUNK = 32
_NPOS = _SEQ_PER_W // _CHUNK
_NCHUNK = _NPOS * _BATCH
_LANES = 16


def _emb_body(ids_hbm, tok_hbm, pos_hbm, out_hbm,
              idx_v, tok0, tok1, posbuf, gsem, wsem):
    wid = lax.axis_index("s") * _NC + lax.axis_index("c")
    seq0 = wid * _SEQ_PER_W

    pltpu.sync_copy(ids_hbm.at[wid], idx_v)

    tokbufs = (tok0, tok1)

    def add_rows(tokbuf):
        def row_body(r, _):
            for j in range(_EMBED // _LANES):
                s = pl.ds(j * _LANES, _LANES)
                tokbuf[r, s] = tokbuf[r, s] + posbuf[r, s]
            return 0
        lax.fori_loop(0, _CHUNK, row_body, 0)

    g0 = pltpu.async_copy(tok_hbm.at[idx_v.at[0]], tok0, gsem)
    gathers = [g0, None]
    writes = [None, None]

    for c in range(_NCHUNK):
        p, b = divmod(c, _BATCH)
        buf = c % 2
        tokbuf = tokbufs[buf]

        if b == 0:
            pltpu.sync_copy(pos_hbm.at[pl.ds(seq0 + p * _CHUNK, _CHUNK)], posbuf)

        gathers[buf].wait()

        if c + 1 < _NCHUNK:
            nbuf = (c + 1) % 2
            if writes[nbuf] is not None:
                writes[nbuf].wait()
                writes[nbuf] = None
            gathers[nbuf] = pltpu.async_copy(
                tok_hbm.at[idx_v.at[c + 1]], tokbufs[nbuf], gsem)

        add_rows(tokbuf)

        row0 = b * _MAX_SEQ + seq0 + p * _CHUNK
        writes[buf] = pltpu.async_copy(
            tokbuf, out_hbm.at[pl.ds(row0, _CHUNK)], wsem)

    for w in writes:
        if w is not None:
            w.wait()


@jax.jit
def _embed(ids, tok_table, pos_table):
    mesh = plsc.VectorSubcoreMesh(core_axis_name="c", subcore_axis_name="s")
    run = functools.partial(
        pl.kernel,
        out_type=jax.ShapeDtypeStruct((_BATCH * _MAX_SEQ, _EMBED), jnp.float32),
        mesh=mesh,
        scratch_types=[
            pltpu.VMEM((_NCHUNK, _CHUNK), jnp.int32),
            pltpu.VMEM((_CHUNK, _EMBED), jnp.float32),
            pltpu.VMEM((_CHUNK, _EMBED), jnp.float32),
            pltpu.VMEM((_CHUNK, _EMBED), jnp.float32),
            pltpu.SemaphoreType.DMA,
            pltpu.SemaphoreType.DMA,
        ],
    )(_emb_body)
    return run(ids, tok_table, pos_table)


def kernel(input_ids, token_table, pos_table):
    ids = (input_ids.astype(jnp.int32)
           .reshape(_BATCH, _NW, _NPOS, _CHUNK)
           .transpose(1, 2, 0, 3)
           .reshape(_NW, _NCHUNK, _CHUNK))
    out = _embed(ids, token_table, pos_table)
    return out.reshape(_BATCH, _MAX_SEQ, _EMBED)
